# Initial kernel scaffold; baseline (speedup 1.0000x reference)
import jax
import jax.numpy as jnp
from jax.experimental import pallas as pl


def kernel(features, edge_idx, W_enc, b_enc, W_dec, b_dec):
    raise NotImplementedError("write your pallas kernel here")


# speedup vs baseline: 14.6548x; 14.6548x over previous
import functools

import jax
import jax.numpy as jnp
from jax import lax
from jax.experimental import pallas as pl
from jax.experimental.pallas import tpu as pltpu
from jax.experimental.pallas import tpu_sc as plsc

NC = 2
NS = 16
LANES = 16
G = 125
SUP = 32
SUPD = 40
HALF = 128
RA = 624
BN = 1000


def _mesh():
    return plsc.VectorSubcoreMesh(core_axis_name="c", subcore_axis_name="s")


def _copy_rows(src_ref, dst_ref, sid, n_rows):
    last = n_rows - (NS - 1) * RA

    @pl.when(sid < NS - 1)
    def _():
        r0 = pl.multiple_of(sid * RA, 8)
        pltpu.sync_copy(src_ref.at[pl.ds(r0, RA)], dst_ref.at[pl.ds(r0, RA)])

    @pl.when(sid == NS - 1)
    def _():
        r0 = (NS - 1) * RA
        pltpu.sync_copy(src_ref.at[pl.ds(r0, last)], dst_ref.at[pl.ds(r0, last)])


def _make_deg(N, E):
    ROWS = E // G
    ROWS_W = ROWS // (NC * NS)

    @functools.partial(
        pl.kernel,
        out_type=[jax.ShapeDtypeStruct((N, HALF), jnp.float32)] * 2,
        mesh=_mesh(),
        scratch_types=[
            pltpu.VMEM((SUPD, G), jnp.int32),
            pltpu.VMEM((G, HALF), jnp.float32),
            pltpu.VMEM_SHARED((N, HALF), jnp.float32),
        ],
    )
    def deg(dst2, zeros_hbm, ones_hbm, out0, out1, idx_v, ones_v, acc_sh):
        cid = lax.axis_index("c")
        sid = lax.axis_index("s")
        wid = cid * NS + sid
        _copy_rows(zeros_hbm, acc_sh, sid, N)
        pltpu.sync_copy(ones_hbm, ones_v)
        plsc.subcore_barrier()

        def outer(j, carry):
            rb = pl.multiple_of(wid * ROWS_W + j * SUPD, 8)
            pltpu.sync_copy(dst2.at[pl.ds(rb, SUPD)], idx_v)

            def inner(k, c2):
                pltpu.sync_copy(ones_v, acc_sh.at[idx_v.at[k]], add=True)
                return c2

            return lax.fori_loop(0, SUPD, inner, carry)

        lax.fori_loop(0, ROWS_W // SUPD, outer, 0)
        plsc.subcore_barrier()

        @pl.when(cid == 0)
        def _():
            _copy_rows(acc_sh, out0, sid, N)

        @pl.when(cid == 1)
        def _():
            _copy_rows(acc_sh, out1, sid, N)

    return deg


def _make_prop(N, E):
    ROWS = E // G
    ROWS_T = ROWS // NS

    @functools.partial(
        pl.kernel,
        out_type=[jax.ShapeDtypeStruct((N, HALF), jnp.float32)] * 2,
        mesh=_mesh(),
        scratch_types=[
            pltpu.VMEM((SUP, G), jnp.int32),
            pltpu.VMEM((SUP, G), jnp.int32),
            pltpu.VMEM((G, HALF), jnp.float32),
            pltpu.VMEM_SHARED((N, HALF), jnp.float32),
            pltpu.SemaphoreType.DMA,
        ],
    )
    def prop(hp0, hp1, src2, dst2, out0, out1, src_v, dst_v, rows_v, acc_sh, sem):
        cid = lax.axis_index("c")
        sid = lax.axis_index("s")

        def run(hp, out):
            _copy_rows(hp, acc_sh, sid, N)
            plsc.subcore_barrier()

            def outer(j, carry):
                rb = pl.multiple_of(sid * ROWS_T + j * SUP, 8)
                pltpu.sync_copy(src2.at[pl.ds(rb, SUP)], src_v)
                pltpu.sync_copy(dst2.at[pl.ds(rb, SUP)], dst_v)

                def inner(k, c2):
                    pltpu.async_copy(hp.at[src_v.at[k]], rows_v, sem).wait()
                    pltpu.sync_copy(rows_v, acc_sh.at[dst_v.at[k]], add=True)
                    return c2

                return lax.fori_loop(0, SUP, inner, carry)

            lax.fori_loop(0, ROWS_T // SUP, outer, 0)
            plsc.subcore_barrier()
            _copy_rows(acc_sh, out, sid, N)

        @pl.when(cid == 0)
        def _():
            run(hp0, out0)

        @pl.when(cid == 1)
        def _():
            run(hp1, out1)

    return prop


def _dinv(d0_ref, d1_ref):
    return lax.rsqrt(1.0 + d0_ref[:, 0:1] + d1_ref[:, 0:1])


def _tc1_body(x_ref, w_ref, d0_ref, d1_ref, o0_ref, o1_ref):
    dinv = _dinv(d0_ref, d1_ref)
    h = jnp.dot(x_ref[...], w_ref[...], precision=lax.Precision.HIGHEST,
                preferred_element_type=jnp.float32)
    o0_ref[...] = h[:, :HALF] * dinv
    o1_ref[...] = h[:, HALF:] * dinv


def _tc2_body(a0_ref, a1_ref, d0_ref, d1_ref, w_ref, b_ref, o0_ref, o1_ref):
    dinv = _dinv(d0_ref, d1_ref)
    b = b_ref[...]
    lat0 = jnp.maximum(a0_ref[...] * dinv + b[:, :HALF], 0.0)
    lat1 = jnp.maximum(a1_ref[...] * dinv + b[:, HALF:], 0.0)
    w = w_ref[...]
    h = (jnp.dot(lat0, w[:HALF, :], precision=lax.Precision.HIGHEST,
                 preferred_element_type=jnp.float32)
         + jnp.dot(lat1, w[HALF:, :], precision=lax.Precision.HIGHEST,
                   preferred_element_type=jnp.float32))
    o0_ref[...] = h[:, :HALF] * dinv
    o1_ref[...] = h[:, HALF:] * dinv


def _tc3_body(a0_ref, a1_ref, d0_ref, d1_ref, b_ref, o_ref):
    dinv = _dinv(d0_ref, d1_ref)
    b = b_ref[...]
    lo = jnp.maximum(a0_ref[...] * dinv + b[:, :HALF], 0.0)
    hi = jnp.maximum(a1_ref[...] * dinv + b[:, HALF:], 0.0)
    o_ref[...] = jnp.concatenate([lo, hi], axis=1)


def _row_spec(bn, d):
    return pl.BlockSpec((bn, d), lambda i: (i, 0))


def _full_spec(shape):
    return pl.BlockSpec(shape, lambda i: (0, 0))


def kernel(features, edge_idx, W_enc, b_enc, W_dec, b_dec):
    N, IN = features.shape
    E = edge_idx.shape[1]
    HID = W_enc.shape[1]

    src = edge_idx[0].astype(jnp.int32).reshape(E // G, G)
    dst = edge_idx[1].astype(jnp.int32).reshape(E // G, G)
    zeros_w = jnp.zeros((N, HALF), jnp.float32)
    ones_w = jnp.ones((G, HALF), jnp.float32)

    d0, d1 = _make_deg(N, E)(dst, zeros_w, ones_w)

    tc1 = pl.pallas_call(
        _tc1_body,
        grid=(N // BN,),
        in_specs=[_row_spec(BN, IN), _full_spec((IN, HID)),
                  _row_spec(BN, HALF), _row_spec(BN, HALF)],
        out_specs=[_row_spec(BN, HALF)] * 2,
        out_shape=[jax.ShapeDtypeStruct((N, HALF), jnp.float32)] * 2,
    )
    hp0, hp1 = tc1(features, W_enc, d0, d1)

    prop = _make_prop(N, E)
    a0, a1 = prop(hp0, hp1, src, dst)

    tc2 = pl.pallas_call(
        _tc2_body,
        grid=(N // BN,),
        in_specs=[_row_spec(BN, HALF), _row_spec(BN, HALF),
                  _row_spec(BN, HALF), _row_spec(BN, HALF),
                  _full_spec((HID, HID)), _full_spec((1, HID))],
        out_specs=[_row_spec(BN, HALF)] * 2,
        out_shape=[jax.ShapeDtypeStruct((N, HALF), jnp.float32)] * 2,
    )
    g0, g1 = tc2(a0, a1, d0, d1, W_dec, b_enc.reshape(1, HID))

    c0, c1 = prop(g0, g1, src, dst)

    tc3 = pl.pallas_call(
        _tc3_body,
        grid=(N // BN,),
        in_specs=[_row_spec(BN, HALF), _row_spec(BN, HALF),
                  _row_spec(BN, HALF), _row_spec(BN, HALF),
                  _full_spec((1, HID))],
        out_specs=_row_spec(BN, HID),
        out_shape=jax.ShapeDtypeStruct((N, HID), jnp.float32),
    )
    return tc3(c0, c1, d0, d1, b_dec.reshape(1, HID))
